# Initial kernel scaffold; baseline (speedup 1.0000x reference)
import jax
import jax.numpy as jnp
from jax.experimental import pallas as pl


def kernel(classifications, regressions, anchors, boxes, labels):
    raise NotImplementedError("write your pallas kernel here")


# speedup vs baseline: 8.2266x; 8.2266x over previous
import functools

import jax
import jax.numpy as jnp
from jax.experimental import pallas as pl
from jax.experimental.pallas import tpu as pltpu

ALPHA = 0.25
GAMMA_POW = 2


def _body(cls_ref, reg_ref, anc_ref, box_ref, lab_ref, out_ref, *, A, TA, T, C, G):
    t = pl.program_id(1)

    anc = anc_ref[...]
    box = box_ref[0]
    lab = lab_ref[0]

    a0 = anc[0:1, :]
    a1 = anc[1:2, :]
    a2 = anc[2:3, :]
    a3 = anc[3:4, :]

    b0 = box[:, 0:1]
    b1 = box[:, 1:2]
    b2 = box[:, 2:3]
    b3 = box[:, 3:4]

    area_b = (b2 - b0) * (b3 - b1)
    iw = jnp.minimum(a2, b2) - jnp.maximum(a0, b0)
    ih = jnp.minimum(a3, b3) - jnp.maximum(a1, b1)
    iw = jnp.clip(iw, 0.0, None)
    ih = jnp.clip(ih, 0.0, None)
    area_a = (a2 - a0) * (a3 - a1)
    ua = area_a + area_b - iw * ih
    ua = jnp.clip(ua, 1e-8, None)
    iou = (iw * ih) / ua

    iou_max = jnp.max(iou, axis=0, keepdims=True)
    g_iota = jax.lax.broadcasted_iota(jnp.int32, (G, TA), 0)
    amax = jnp.min(jnp.where(iou == iou_max, g_iota, G), axis=0, keepdims=True)
    onehot = g_iota == amax

    ab0 = jnp.sum(jnp.where(onehot, b0, 0.0), axis=0, keepdims=True)
    ab1 = jnp.sum(jnp.where(onehot, b1, 0.0), axis=0, keepdims=True)
    ab2 = jnp.sum(jnp.where(onehot, b2, 0.0), axis=0, keepdims=True)
    ab3 = jnp.sum(jnp.where(onehot, b3, 0.0), axis=0, keepdims=True)
    alab = jnp.sum(jnp.where(onehot, lab, 0), axis=0, keepdims=True)

    a_idx = t * TA + jax.lax.broadcasted_iota(jnp.int32, (1, TA), 1)
    inb = a_idx < A
    pos = jnp.logical_and(iou_max >= 0.5, inb)
    valid = jnp.logical_and(
        jnp.logical_or(iou_max < 0.4, iou_max >= 0.5), inb)

    num_pos_p = jnp.sum(jnp.where(pos, 1.0, 0.0))

    a_w = a2 - a0
    a_h = a3 - a1
    a_cx = a0 + 0.5 * a_w
    a_cy = a1 + 0.5 * a_h
    gw0 = ab2 - ab0
    gh0 = ab3 - ab1
    gcx = ab0 + 0.5 * gw0
    gcy = ab1 + 0.5 * gh0
    gw = jnp.clip(gw0, 1.0, None)
    gh = jnp.clip(gh0, 1.0, None)
    t0 = ((gcx - a_cx) / a_w) / 0.1
    t1 = ((gcy - a_cy) / a_h) / 0.1
    t2 = jnp.log(gw / a_w) / 0.2
    t3 = jnp.log(gh / a_h) / 0.2

    reg = reg_ref[0]
    reg_p = 0.0
    for k, tk in enumerate((t0, t1, t2, t3)):
        d = jnp.abs(tk - reg[k:k + 1, :])
        rl = jnp.where(d <= 1.0 / 9.0, 0.5 * 9.0 * d * d, d - 0.5 / 9.0)
        reg_p = reg_p + jnp.sum(jnp.where(pos, rl, 0.0))

    iou_max_col = iou_max.T
    alab_col = alab.T
    inb_col = (t * TA + jax.lax.broadcasted_iota(jnp.int32, (TA, 1), 0)) < A
    pos_col = jnp.logical_and(iou_max_col >= 0.5, inb_col)
    valid_col = jnp.logical_and(
        jnp.logical_or(iou_max_col < 0.4, iou_max_col >= 0.5), inb_col)

    cls = jnp.clip(cls_ref[0], 1e-4, 1.0 - 1e-4)
    neg_term = (1.0 - ALPHA) * cls * cls * (-jnp.log(1.0 - cls))
    s_neg = jnp.sum(jnp.where(valid_col, neg_term, 0.0))

    c_iota = jax.lax.broadcasted_iota(jnp.int32, (TA, C), 1)
    cg = jnp.sum(jnp.where(c_iota == alab_col, cls, 0.0), axis=1, keepdims=True)
    cg = jnp.clip(cg, 1e-4, 1.0 - 1e-4)
    pos_t = ALPHA * (1.0 - cg) * (1.0 - cg) * (-jnp.log(cg))
    neg_t = (1.0 - ALPHA) * cg * cg * (-jnp.log(1.0 - cg))
    corr = jnp.sum(jnp.where(pos_col, pos_t - neg_t, 0.0))

    cls_p = s_neg + corr

    l_iota = jax.lax.broadcasted_iota(jnp.int32, (1, 1, 128), 2)
    vec = (jnp.where(l_iota == 0, cls_p, 0.0)
           + jnp.where(l_iota == 1, reg_p, 0.0)
           + jnp.where(l_iota == 2, num_pos_p, 0.0))

    @pl.when(t == 0)
    def _init():
        out_ref[...] = vec

    @pl.when(t > 0)
    def _acc():
        out_ref[...] = out_ref[...] + vec

    @pl.when(t == T - 1)
    def _fin():
        acc = out_ref[...]
        npos = jnp.maximum(acc[0, 0, 2], 1.0)
        cls_l = acc[0, 0, 0] / npos
        reg_l = acc[0, 0, 1] / (npos * 4.0)
        out_ref[...] = (jnp.where(l_iota == 0, cls_l, 0.0)
                        + jnp.where(l_iota == 1, reg_l, 0.0)
                        + jnp.where(l_iota == 2, acc[0, 0, 2], 0.0))


@jax.jit
def kernel(classifications, regressions, anchors, boxes, labels):
    B, A, C = classifications.shape
    G = boxes.shape[1]
    TA = 2048
    T = (A + TA - 1) // TA

    reg_t = regressions.transpose(0, 2, 1)
    anc_t = anchors[0].T
    lab3 = labels.astype(jnp.int32)[..., None]

    body = functools.partial(_body, A=A, TA=TA, T=T, C=C, G=G)
    out = pl.pallas_call(
        body,
        grid=(B, T),
        in_specs=[
            pl.BlockSpec((1, TA, C), lambda j, t: (j, t, 0)),
            pl.BlockSpec((1, 4, TA), lambda j, t: (j, 0, t)),
            pl.BlockSpec((4, TA), lambda j, t: (0, t)),
            pl.BlockSpec((1, G, 4), lambda j, t: (j, 0, 0)),
            pl.BlockSpec((1, G, 1), lambda j, t: (j, 0, 0)),
        ],
        out_specs=pl.BlockSpec((1, 1, 128), lambda j, t: (j, 0, 0)),
        out_shape=jax.ShapeDtypeStruct((B, 1, 128), jnp.float32),
        compiler_params=pltpu.CompilerParams(
            dimension_semantics=("arbitrary", "arbitrary")),
    )(classifications, reg_t, anc_t, boxes, lab3)

    cls_loss = jnp.mean(out[:, 0, 0:1], axis=0)
    reg_loss = jnp.mean(out[:, 0, 1:2], axis=0)
    return cls_loss, reg_loss


# speedup vs baseline: 12.6933x; 1.5430x over previous
import functools

import jax
import jax.numpy as jnp
from jax.experimental import pallas as pl
from jax.experimental.pallas import tpu as pltpu

ALPHA = 0.25
GAMMA_POW = 2


def _body(cls_ref, reg_ref, anc_ref, box_ref, lab_ref, out_ref, *, A, TA, T, C, G):
    t = pl.program_id(1)

    anc = anc_ref[...]
    box = box_ref[0]
    lab = lab_ref[0]

    a0 = anc[0:1, :]
    a1 = anc[1:2, :]
    a2 = anc[2:3, :]
    a3 = anc[3:4, :]

    b0 = box[:, 0:1]
    b1 = box[:, 1:2]
    b2 = box[:, 2:3]
    b3 = box[:, 3:4]

    area_b = (b2 - b0) * (b3 - b1)
    iw = jnp.minimum(a2, b2) - jnp.maximum(a0, b0)
    ih = jnp.minimum(a3, b3) - jnp.maximum(a1, b1)
    iw = jnp.clip(iw, 0.0, None)
    ih = jnp.clip(ih, 0.0, None)
    area_a = (a2 - a0) * (a3 - a1)
    ua = area_a + area_b - iw * ih
    ua = jnp.clip(ua, 1e-8, None)
    iou = (iw * ih) / ua

    iou_max = jnp.max(iou, axis=0, keepdims=True)
    g_iota = jax.lax.broadcasted_iota(jnp.int32, (G, TA), 0)
    amax = jnp.min(jnp.where(iou == iou_max, g_iota, G), axis=0, keepdims=True)
    onehot = g_iota == amax

    onehotf = jnp.where(onehot, 1.0, 0.0)

    a_idx = t * TA + jax.lax.broadcasted_iota(jnp.int32, (1, TA), 1)
    inb = a_idx < A
    pos = jnp.logical_and(iou_max >= 0.5, inb)
    valid = jnp.logical_and(
        jnp.logical_or(iou_max < 0.4, iou_max >= 0.5), inb)

    p = b2 - b0
    q = b3 - b1
    r = 0.5 * (b0 + b2)
    s = 0.5 * (b1 + b3)
    boxmix = jnp.concatenate([p, q, r, s], axis=1)
    oh_bf = onehotf.astype(jnp.bfloat16)
    dn_g = (((0,), (0,)), ((), ()))
    bm1 = boxmix.astype(jnp.bfloat16)
    bmr1 = boxmix - bm1.astype(jnp.float32)
    bm2 = bmr1.astype(jnp.bfloat16)
    bm3 = (bmr1 - bm2.astype(jnp.float32)).astype(jnp.bfloat16)
    asg = (jax.lax.dot_general(bm1, oh_bf, dn_g,
                               preferred_element_type=jnp.float32)
           + jax.lax.dot_general(bm2, oh_bf, dn_g,
                                 preferred_element_type=jnp.float32)
           + jax.lax.dot_general(bm3, oh_bf, dn_g,
                                 preferred_element_type=jnp.float32))
    gw0 = asg[0:1, :]
    gh0 = asg[1:2, :]
    gcx = asg[2:3, :]
    gcy = asg[3:4, :]

    a_w = a2 - a0
    a_h = a3 - a1
    a_cx = a0 + 0.5 * a_w
    a_cy = a1 + 0.5 * a_h
    gw = jnp.clip(gw0, 1.0, None)
    gh = jnp.clip(gh0, 1.0, None)
    t0 = ((gcx - a_cx) / a_w) / 0.1
    t1 = ((gcy - a_cy) / a_h) / 0.1
    t2 = jnp.log(gw / a_w) / 0.2
    t3 = jnp.log(gh / a_h) / 0.2

    reg = reg_ref[0]
    regacc = jnp.zeros((1, TA), jnp.float32)
    for k, tk in enumerate((t0, t1, t2, t3)):
        d = jnp.abs(tk - reg[k:k + 1, :])
        rl = jnp.where(d <= 1.0 / 9.0, 0.5 * 9.0 * d * d, d - 0.5 / 9.0)
        regacc = regacc + jnp.where(pos, rl, 0.0)

    craw = cls_ref[0]
    cls = jnp.where(craw > 1e-4, craw, 1e-4)
    cls = jnp.where(cls < 1.0 - 1e-4, cls, 1.0 - 1e-4)
    neg_term = (1.0 - ALPHA) * cls * cls * (-jnp.log(1.0 - cls))

    validf_bf = jnp.where(valid, 1.0, 0.0).astype(jnp.bfloat16)
    neg_bf = neg_term.astype(jnp.bfloat16)
    sneg_vec = jax.lax.dot_general(validf_bf, neg_bf, (((1,), (0,)), ((), ())),
                                   preferred_element_type=jnp.float32)

    c_iota_g = jax.lax.broadcasted_iota(jnp.int32, (G, C), 1)
    Lf_bf = jnp.where(c_iota_g == lab, 1.0, 0.0).astype(jnp.bfloat16)
    cls_bf = cls.astype(jnp.bfloat16)
    M_T = jax.lax.dot_general(Lf_bf, cls_bf, (((1,), (1,)), ((), ())),
                              preferred_element_type=jnp.float32)
    cgv = jnp.sum(onehotf * M_T, axis=0, keepdims=True)
    cg = jnp.where(cgv > 1e-4, cgv, 1e-4)
    cg = jnp.where(cg < 1.0 - 1e-4, cg, 1.0 - 1e-4)
    pos_t = ALPHA * (1.0 - cg) * (1.0 - cg) * (-jnp.log(cg))
    neg_t = (1.0 - ALPHA) * cg * cg * (-jnp.log(1.0 - cg))
    corrv = jnp.where(pos, pos_t - neg_t, 0.0)

    nposv = jnp.where(pos, 1.0, 0.0)

    def fold128(v):
        acc = v[:, 0:128]
        for kk in range(1, TA // 128):
            acc = acc + v[:, kk * 128:(kk + 1) * 128]
        return acc

    sneg_pad = jnp.concatenate(
        [sneg_vec, jnp.zeros((1, 128 - C), jnp.float32)], axis=1)
    vec = jnp.concatenate(
        [sneg_pad, fold128(corrv), fold128(regacc), fold128(nposv)],
        axis=0)[None]

    @pl.when(t == 0)
    def _init():
        out_ref[...] = vec

    @pl.when(t > 0)
    def _acc():
        out_ref[...] = out_ref[...] + vec

    @pl.when(t == T - 1)
    def _fin():
        acc = out_ref[...]
        cls_sum = jnp.sum(acc[0, 0:2, :])
        reg_sum = jnp.sum(acc[0, 2:3, :])
        npos = jnp.maximum(jnp.sum(acc[0, 3:4, :]), 1.0)
        cls_l = cls_sum / npos
        reg_l = reg_sum / (npos * 4.0)
        l_iota = jax.lax.broadcasted_iota(jnp.int32, (1, 4, 128), 2)
        r_iota = jax.lax.broadcasted_iota(jnp.int32, (1, 4, 128), 1)
        out_ref[...] = (
            jnp.where(jnp.logical_and(r_iota == 0, l_iota == 0), cls_l, 0.0)
            + jnp.where(jnp.logical_and(r_iota == 0, l_iota == 1), reg_l, 0.0))


@jax.jit
def kernel(classifications, regressions, anchors, boxes, labels):
    B, A, C = classifications.shape
    G = boxes.shape[1]
    TA = 2048
    T = (A + TA - 1) // TA

    reg_t = regressions.transpose(0, 2, 1)
    anc_t = anchors[0].T
    lab3 = labels.astype(jnp.int32)[..., None]

    body = functools.partial(_body, A=A, TA=TA, T=T, C=C, G=G)
    out = pl.pallas_call(
        body,
        grid=(B, T),
        in_specs=[
            pl.BlockSpec((1, TA, C), lambda j, t: (j, t, 0)),
            pl.BlockSpec((1, 4, TA), lambda j, t: (j, 0, t)),
            pl.BlockSpec((4, TA), lambda j, t: (0, t)),
            pl.BlockSpec((1, G, 4), lambda j, t: (j, 0, 0)),
            pl.BlockSpec((1, G, 1), lambda j, t: (j, 0, 0)),
        ],
        out_specs=pl.BlockSpec((1, 4, 128), lambda j, t: (j, 0, 0)),
        out_shape=jax.ShapeDtypeStruct((B, 4, 128), jnp.float32),
        compiler_params=pltpu.CompilerParams(
            dimension_semantics=("arbitrary", "arbitrary")),
    )(classifications, reg_t, anc_t, boxes, lab3)

    cls_loss = jnp.mean(out[:, 0, 0:1], axis=0)
    reg_loss = jnp.mean(out[:, 0, 1:2], axis=0)
    return cls_loss, reg_loss


# speedup vs baseline: 13.9722x; 1.1008x over previous
import functools

import jax
import jax.numpy as jnp
from jax.experimental import pallas as pl
from jax.experimental.pallas import tpu as pltpu

ALPHA = 0.25
GAMMA_POW = 2


def _body(cls_ref, reg_ref, anc_ref, box_ref, lab_ref, out_ref, *, A, TA, T, C, G):
    t = pl.program_id(1)

    anc = anc_ref[...]
    box = box_ref[0]
    lab = lab_ref[0]

    a0 = anc[0:1, :]
    a1 = anc[1:2, :]
    a2 = anc[2:3, :]
    a3 = anc[3:4, :]

    b0 = box[:, 0:1]
    b1 = box[:, 1:2]
    b2 = box[:, 2:3]
    b3 = box[:, 3:4]

    area_b = (b2 - b0) * (b3 - b1)
    iw = jnp.minimum(a2, b2) - jnp.maximum(a0, b0)
    ih = jnp.minimum(a3, b3) - jnp.maximum(a1, b1)
    iw = jnp.clip(iw, 0.0, None)
    ih = jnp.clip(ih, 0.0, None)
    area_a = (a2 - a0) * (a3 - a1)
    inter = iw * ih
    ua = area_a + area_b - inter
    iou = inter / ua

    iou_max = jnp.max(iou, axis=0, keepdims=True)
    g_iota = jax.lax.broadcasted_iota(jnp.int32, (G, TA), 0)
    amax = jnp.min(jnp.where(iou == iou_max, g_iota, G), axis=0, keepdims=True)
    onehot = g_iota == amax

    a_idx = t * TA + jax.lax.broadcasted_iota(jnp.int32, (1, TA), 1)
    inb = a_idx < A
    pos = jnp.logical_and(iou_max >= 0.5, inb)
    valid = jnp.logical_and(
        jnp.logical_or(iou_max < 0.4, iou_max >= 0.5), inb)

    ponehot = jnp.where(jnp.logical_and(onehot, pos), 1.0, 0.0)

    p = b2 - b0
    q = b3 - b1
    r = 0.5 * (b0 + b2)
    s = 0.5 * (b1 + b3)
    boxmix = jnp.concatenate([p, q, r, s], axis=1)
    oh_bf = ponehot.astype(jnp.bfloat16)
    dn_g = (((0,), (0,)), ((), ()))
    bm1 = boxmix.astype(jnp.bfloat16)
    bmr1 = boxmix - bm1.astype(jnp.float32)
    bm2 = bmr1.astype(jnp.bfloat16)
    bm3 = (bmr1 - bm2.astype(jnp.float32)).astype(jnp.bfloat16)
    asg = (jax.lax.dot_general(bm1, oh_bf, dn_g,
                               preferred_element_type=jnp.float32)
           + jax.lax.dot_general(bm2, oh_bf, dn_g,
                                 preferred_element_type=jnp.float32)
           + jax.lax.dot_general(bm3, oh_bf, dn_g,
                                 preferred_element_type=jnp.float32))
    gw0 = asg[0:1, :]
    gh0 = asg[1:2, :]
    gcx = asg[2:3, :]
    gcy = asg[3:4, :]

    a_w = a2 - a0
    a_h = a3 - a1
    a_cx = a0 + 0.5 * a_w
    a_cy = a1 + 0.5 * a_h
    gw = jnp.clip(gw0, 1.0, None)
    gh = jnp.clip(gh0, 1.0, None)
    t0 = ((gcx - a_cx) / a_w) / 0.1
    t1 = ((gcy - a_cy) / a_h) / 0.1
    t2 = jnp.log(gw / a_w) / 0.2
    t3 = jnp.log(gh / a_h) / 0.2

    reg = reg_ref[0]
    regacc = jnp.zeros((1, TA), jnp.float32)
    for k, tk in enumerate((t0, t1, t2, t3)):
        d = jnp.abs(tk - reg[k:k + 1, :])
        rl = jnp.where(d <= 1.0 / 9.0, 0.5 * 9.0 * d * d, d - 0.5 / 9.0)
        regacc = regacc + jnp.where(pos, rl, 0.0)

    craw = cls_ref[0]
    cls = jnp.where(craw > 1e-4, craw, 1e-4)
    cls = jnp.where(cls < 1.0 - 1e-4, cls, 1.0 - 1e-4)
    om = 1.0 - cls
    cls_bf = cls.astype(jnp.bfloat16)
    om_bf = om.astype(jnp.bfloat16)
    neg_bf = (jnp.bfloat16(1.0 - ALPHA) * cls_bf * cls_bf
              * (-jnp.log(om_bf)))

    validf_bf = jnp.where(valid, 1.0, 0.0).astype(jnp.bfloat16)
    sneg_vec = jax.lax.dot_general(validf_bf, neg_bf, (((1,), (0,)), ((), ())),
                                   preferred_element_type=jnp.float32)

    pos_bf = (jnp.bfloat16(ALPHA) * om_bf * om_bf
              * (-jnp.log(cls_bf)))
    delta_bf = pos_bf - neg_bf
    ponehot_bf = ponehot.astype(jnp.bfloat16)
    D = jax.lax.dot_general(ponehot_bf, delta_bf, (((1,), (0,)), ((), ())),
                            preferred_element_type=jnp.float32)
    c_iota_g = jax.lax.broadcasted_iota(jnp.int32, (G, C), 1)
    Lf = jnp.where(c_iota_g == lab, 1.0, 0.0)
    corr_vec = jnp.sum(Lf * D, axis=0, keepdims=True)

    nposv = jnp.where(pos, 1.0, 0.0)

    def fold128(v):
        acc = v[:, 0:128]
        for kk in range(1, TA // 128):
            acc = acc + v[:, kk * 128:(kk + 1) * 128]
        return acc

    cls_row = jnp.concatenate(
        [sneg_vec + corr_vec, jnp.zeros((1, 128 - C), jnp.float32)], axis=1)
    vec = jnp.concatenate(
        [cls_row, jnp.zeros((1, 128), jnp.float32), fold128(regacc),
         fold128(nposv)], axis=0)[None]

    @pl.when(t == 0)
    def _init():
        out_ref[...] = vec

    @pl.when(t > 0)
    def _acc():
        out_ref[...] = out_ref[...] + vec

    @pl.when(t == T - 1)
    def _fin():
        acc = out_ref[...]
        cls_sum = jnp.sum(acc[0, 0:2, :])
        reg_sum = jnp.sum(acc[0, 2:3, :])
        npos = jnp.maximum(jnp.sum(acc[0, 3:4, :]), 1.0)
        cls_l = cls_sum / npos
        reg_l = reg_sum / (npos * 4.0)
        l_iota = jax.lax.broadcasted_iota(jnp.int32, (1, 4, 128), 2)
        r_iota = jax.lax.broadcasted_iota(jnp.int32, (1, 4, 128), 1)
        out_ref[...] = (
            jnp.where(jnp.logical_and(r_iota == 0, l_iota == 0), cls_l, 0.0)
            + jnp.where(jnp.logical_and(r_iota == 0, l_iota == 1), reg_l, 0.0))


@jax.jit
def kernel(classifications, regressions, anchors, boxes, labels):
    B, A, C = classifications.shape
    G = boxes.shape[1]
    TA = 4096
    T = (A + TA - 1) // TA

    reg_t = regressions.transpose(0, 2, 1)
    anc_t = anchors[0].T
    lab3 = labels.astype(jnp.int32)[..., None]

    body = functools.partial(_body, A=A, TA=TA, T=T, C=C, G=G)
    out = pl.pallas_call(
        body,
        grid=(B, T),
        in_specs=[
            pl.BlockSpec((1, TA, C), lambda j, t: (j, t, 0)),
            pl.BlockSpec((1, 4, TA), lambda j, t: (j, 0, t)),
            pl.BlockSpec((4, TA), lambda j, t: (0, t)),
            pl.BlockSpec((1, G, 4), lambda j, t: (j, 0, 0)),
            pl.BlockSpec((1, G, 1), lambda j, t: (j, 0, 0)),
        ],
        out_specs=pl.BlockSpec((1, 4, 128), lambda j, t: (j, 0, 0)),
        out_shape=jax.ShapeDtypeStruct((B, 4, 128), jnp.float32),
        compiler_params=pltpu.CompilerParams(
            dimension_semantics=("arbitrary", "arbitrary")),
    )(classifications, reg_t, anc_t, boxes, lab3)

    cls_loss = jnp.mean(out[:, 0, 0:1], axis=0)
    reg_loss = jnp.mean(out[:, 0, 1:2], axis=0)
    return cls_loss, reg_loss
